# Initial kernel scaffold; baseline (speedup 1.0000x reference)
import jax
import jax.numpy as jnp
from jax.experimental import pallas as pl


def kernel(x, W, b):
    raise NotImplementedError("write your pallas kernel here")


# speedup vs baseline: 5.6475x; 5.6475x over previous
import functools

import jax
import jax.numpy as jnp
from jax import lax
from jax.experimental import pallas as pl
from jax.experimental.pallas import tpu as pltpu
from jax.experimental.pallas import tpu_sc as plsc

_BATCH = 2
_SEQ = 8192
_DIM = 4096
_BLK = 512
_TOPK = 64
_NBLK = _BATCH * (_SEQ // _BLK)
_NSEL = _NBLK * _TOPK


def _select_body(x_ref, w_ref, b_ref, out_ref):
    g = pl.program_id(0)
    xb = x_ref[0]
    w = w_ref[...]
    scores = jnp.dot(xb, w, preferred_element_type=jnp.float32) + b_ref[...]
    s_col = scores
    s_row = jnp.transpose(scores)
    gt = s_row > s_col
    eq = s_row == s_col
    jlt = (lax.broadcasted_iota(jnp.int32, (_BLK, _BLK), 1)
           < lax.broadcasted_iota(jnp.int32, (_BLK, _BLK), 0))
    rank = jnp.sum((gt | (eq & jlt)).astype(jnp.int32), axis=1, keepdims=True)
    r_row = lax.broadcasted_iota(jnp.int32, (1, _TOPK), 1)
    sel = (rank == r_row).astype(jnp.int32)
    tok = lax.broadcasted_iota(jnp.int32, (_BLK, 1), 0) + g * _BLK
    out_ref[0] = jnp.sum(sel * tok, axis=0, keepdims=True)


def _select_indices(x, W, b):
    xg = x.reshape(_NBLK, _BLK, _DIM)
    w2 = W.reshape(_DIM, 1)
    b2 = b.reshape(1, 1)
    idx = pl.pallas_call(
        _select_body,
        grid=(_NBLK,),
        in_specs=[
            pl.BlockSpec((1, _BLK, _DIM), lambda i: (i, 0, 0)),
            pl.BlockSpec((_DIM, 1), lambda i: (0, 0)),
            pl.BlockSpec((1, 1), lambda i: (0, 0)),
        ],
        out_specs=pl.BlockSpec((1, 1, _TOPK), lambda i: (i, 0, 0)),
        out_shape=jax.ShapeDtypeStruct((_NBLK, 1, _TOPK), jnp.int32),
    )(xg, w2, b2)
    return idx.reshape(_NSEL)


def _gather(xflat, idx):
    rows_per_sub = _NSEL // 32
    chunk = 16

    @functools.partial(
        pl.kernel,
        mesh=plsc.VectorSubcoreMesh(core_axis_name="c", subcore_axis_name="s"),
        out_type=jax.ShapeDtypeStruct((_NSEL, _DIM), jnp.float32),
        scratch_types=[
            pltpu.VMEM((chunk,), jnp.int32),
            pltpu.VMEM((chunk, _DIM), jnp.float32),
            pltpu.SemaphoreType.DMA,
        ],
    )
    def gk(x_hbm, idx_hbm, out_hbm, idx_v, rows_v, sem):
        wid = lax.axis_index("s") * 2 + lax.axis_index("c")
        base = wid * rows_per_sub
        for c in range(rows_per_sub // chunk):
            off = base + c * chunk
            pltpu.sync_copy(idx_hbm.at[pl.ds(off, chunk)], idx_v)
            pltpu.async_copy(x_hbm.at[idx_v], rows_v, sem).wait()
            pltpu.sync_copy(rows_v, out_hbm.at[pl.ds(off, chunk)])

    return gk(xflat, idx)


def kernel(x, W, b):
    idx = _select_indices(x, W, b)
    xflat = x.reshape(_BATCH * _SEQ, _DIM)
    out = _gather(xflat, idx)
    return out.reshape(_BATCH, _NSEL // _BATCH, _DIM)


# speedup vs baseline: 5.7273x; 1.0141x over previous
import functools

import jax
import jax.numpy as jnp
from jax import lax
from jax.experimental import pallas as pl
from jax.experimental.pallas import tpu as pltpu
from jax.experimental.pallas import tpu_sc as plsc

_BATCH = 2
_SEQ = 8192
_DIM = 4096
_BLK = 512
_TOPK = 64
_NBLK = _BATCH * (_SEQ // _BLK)
_NSEL = _NBLK * _TOPK


def _select_body(x_ref, w_ref, b_ref, out_ref):
    g = pl.program_id(0)
    xb = x_ref[0]
    w = w_ref[...]
    scores = jnp.dot(xb, w, preferred_element_type=jnp.float32) + b_ref[...]
    s_col = scores
    s_row = jnp.transpose(scores)
    gt = s_row > s_col
    eq = s_row == s_col
    jlt = (lax.broadcasted_iota(jnp.int32, (_BLK, _BLK), 1)
           < lax.broadcasted_iota(jnp.int32, (_BLK, _BLK), 0))
    rank = jnp.sum((gt | (eq & jlt)).astype(jnp.int32), axis=1, keepdims=True)
    r_row = lax.broadcasted_iota(jnp.int32, (1, _TOPK), 1)
    sel = (rank == r_row).astype(jnp.int32)
    tok = lax.broadcasted_iota(jnp.int32, (_BLK, 1), 0) + g * _BLK
    out_ref[0] = jnp.sum(sel * tok, axis=0, keepdims=True)


def _select_indices(x, W, b):
    xg = x.reshape(_NBLK, _BLK, _DIM)
    w2 = W.reshape(_DIM, 1)
    b2 = b.reshape(1, 1)
    idx = pl.pallas_call(
        _select_body,
        grid=(_NBLK,),
        in_specs=[
            pl.BlockSpec((1, _BLK, _DIM), lambda i: (i, 0, 0)),
            pl.BlockSpec((_DIM, 1), lambda i: (0, 0)),
            pl.BlockSpec((1, 1), lambda i: (0, 0)),
        ],
        out_specs=pl.BlockSpec((1, 1, _TOPK), lambda i: (i, 0, 0)),
        out_shape=jax.ShapeDtypeStruct((_NBLK, 1, _TOPK), jnp.int32),
    )(xg, w2, b2)
    return idx.reshape(_NSEL)


def _gather(xflat, idx):
    rows_per_sub = _NSEL // 32
    chunk = 8
    nchunks = rows_per_sub // chunk

    @functools.partial(
        pl.kernel,
        mesh=plsc.VectorSubcoreMesh(core_axis_name="c", subcore_axis_name="s"),
        out_type=jax.ShapeDtypeStruct((_NSEL, _DIM), jnp.float32),
        scratch_types=[
            pltpu.VMEM((chunk,), jnp.int32),
            pltpu.VMEM((chunk,), jnp.int32),
            pltpu.VMEM((chunk, _DIM), jnp.float32),
            pltpu.VMEM((chunk, _DIM), jnp.float32),
            pltpu.SemaphoreType.DMA,
            pltpu.SemaphoreType.DMA,
            pltpu.SemaphoreType.DMA,
            pltpu.SemaphoreType.DMA,
        ],
    )
    def gk(x_hbm, idx_hbm, out_hbm, i0, i1, r0, r1, g0, g1, w0, w1):
        wid = lax.axis_index("s") * 2 + lax.axis_index("c")
        base = wid * rows_per_sub
        idx_v, rows_v = [i0, i1], [r0, r1]
        gsem, wsem = [g0, g1], [w0, w1]
        gops = [None, None]
        wops = [None, None]
        pltpu.sync_copy(idx_hbm.at[pl.ds(base, chunk)], idx_v[0])
        gops[0] = pltpu.async_copy(x_hbm.at[idx_v[0]], rows_v[0], gsem[0])
        for c in range(nchunks):
            b = c % 2
            nb = (c + 1) % 2
            if c + 1 < nchunks:
                if wops[nb] is not None:
                    wops[nb].wait()
                off = base + (c + 1) * chunk
                pltpu.sync_copy(idx_hbm.at[pl.ds(off, chunk)], idx_v[nb])
                gops[nb] = pltpu.async_copy(x_hbm.at[idx_v[nb]], rows_v[nb], gsem[nb])
            gops[b].wait()
            wops[b] = pltpu.async_copy(
                rows_v[b], out_hbm.at[pl.ds(base + c * chunk, chunk)], wsem[b])
        wops[0].wait()
        wops[1].wait()

    return gk(xflat, idx)


def kernel(x, W, b):
    idx = _select_indices(x, W, b)
    xflat = x.reshape(_BATCH * _SEQ, _DIM)
    out = _gather(xflat, idx)
    return out.reshape(_BATCH, _NSEL // _BATCH, _DIM)
